# baseline (speedup 1.0000x reference)
import jax
import jax.numpy as jnp
from jax import lax
from jax.experimental import pallas as pl
from jax.experimental.pallas import tpu as pltpu
from jax.experimental.pallas import tpu_sc as plsc

N = 10000
E = 160000
D = 256
H = 128

NTILES = 16
BINS = 80
BB = 128
PASSES = BINS // NTILES
GP = BINS * BB

NPT = 624
NTAIL = N - NTILES * NPT

SCAN_T = E // 32
SCAN_BUF = 5008
STCAP = 5152
STW = STCAP * 3

MBLK = 1000
GRID = N // MBLK


def _sc_mesh():
    return plsc.VectorSubcoreMesh(core_axis_name="c", subcore_axis_name="s")


def _prep_body(src_h, dst_h, ew_h, degp, bpk, soff, scnt,
               srcb, dstb, ewb, degpart, hist, off, off0, stage):
    c = lax.axis_index("c")
    s = lax.axis_index("s")
    wid = s * 2 + c
    base = wid * SCAN_T

    zi = jnp.zeros((16,), jnp.int32)
    zf = jnp.zeros((16,), jnp.float32)
    ones = jnp.full((16,), 1, jnp.int32)
    lane = lax.iota(jnp.int32, 16)

    srcb[pl.ds(SCAN_BUF - 16, 16)] = zi
    dstb[pl.ds(SCAN_BUF - 16, 16)] = zi
    ewb[pl.ds(SCAN_BUF - 16, 16)] = zf
    pltpu.sync_copy(src_h.at[pl.ds(base, SCAN_T)], srcb.at[pl.ds(0, SCAN_T)])
    pltpu.sync_copy(dst_h.at[pl.ds(base, SCAN_T)], dstb.at[pl.ds(0, SCAN_T)])
    pltpu.sync_copy(ew_h.at[pl.ds(base, SCAN_T)], ewb.at[pl.ds(0, SCAN_T)])

    def _zero(i, _):
        degpart[pl.ds(i * 16, 16)] = zf
        return _
    lax.fori_loop(0, N // 16, _zero, None)

    def _dscat(k, _):
        idx = dstb[pl.ds(k * 16, 16)]
        val = ewb[pl.ds(k * 16, 16)]
        mask = (k * 16 + lane) < SCAN_T
        plsc.addupdate_scatter(degpart, [idx], val, mask=mask)
        return _
    lax.fori_loop(0, SCAN_BUF // 16, _dscat, None)
    pltpu.sync_copy(degpart, degp.at[pl.ds(wid * N, N)])

    for g in range(BINS // 16):
        hist[pl.ds(g * 16, 16)] = zi

    def _hist(k, _):
        binv = lax.shift_right_logical(srcb[pl.ds(k * 16, 16)], 7)
        mask = (k * 16 + lane) < SCAN_T
        plsc.addupdate_scatter(hist, [binv], ones, mask=mask)
        return _
    lax.fori_loop(0, SCAN_BUF // 16, _hist, None)

    carry = jnp.int32(0)
    for g in range(BINS // 16):
        hv = hist[pl.ds(g * 16, 16)]
        inc = plsc.cumsum(hv)
        excl = inc - hv + carry
        off[pl.ds(g * 16, 16)] = excl
        off0[pl.ds(g * 16, 16)] = excl
        carry = carry + inc[15]

    def _zstage(i, _):
        stage[pl.ds(i * 16, 16)] = zi
        return _
    lax.fori_loop(0, STW // 16, _zstage, None)

    def _place(k, _):
        srcv = srcb[pl.ds(k * 16, 16)]
        dstv = dstb[pl.ds(k * 16, 16)]
        ewv = ewb[pl.ds(k * 16, 16)]
        binv = lax.shift_right_logical(srcv, 7)
        valid = (k * 16 + lane) < SCAN_T
        offg = plsc.load_gather(off, [binv])
        cntv, _last = plsc.scan_count(binv, valid)
        pos = offg + cntv - 1
        ps = pos + lax.shift_left(lax.shift_right_logical(pos, 4), 5)
        plsc.store_scatter(stage, [ps], srcv, mask=valid)
        plsc.store_scatter(stage, [ps + 16], dstv, mask=valid)
        plsc.store_scatter(stage, [ps + 32], plsc.bitcast(ewv, jnp.int32),
                           mask=valid)
        plsc.addupdate_scatter(off, [binv], ones, mask=valid)
        return _
    lax.fori_loop(0, SCAN_BUF // 16, _place, None)

    pltpu.sync_copy(stage, bpk.at[pl.ds(wid * STW, STW)])
    pltpu.sync_copy(off0, soff.at[pl.ds(wid * BINS, BINS)])
    pltpu.sync_copy(hist, scnt.at[pl.ds(wid * BINS, BINS)])


def _prep(src, dst, ew):
    f = pl.kernel(
        _prep_body,
        out_type=[jax.ShapeDtypeStruct((32 * N,), jnp.float32),
                  jax.ShapeDtypeStruct((32 * STW,), jnp.int32),
                  jax.ShapeDtypeStruct((32 * BINS,), jnp.int32),
                  jax.ShapeDtypeStruct((32 * BINS,), jnp.int32)],
        mesh=_sc_mesh(),
        compiler_params=pltpu.CompilerParams(needs_layout_passes=False),
        scratch_types=[
            pltpu.VMEM((SCAN_BUF,), jnp.int32),
            pltpu.VMEM((SCAN_BUF,), jnp.int32),
            pltpu.VMEM((SCAN_BUF,), jnp.float32),
            pltpu.VMEM((N,), jnp.float32),
            pltpu.VMEM((BINS,), jnp.int32),
            pltpu.VMEM((BINS,), jnp.int32),
            pltpu.VMEM((BINS,), jnp.int32),
            pltpu.VMEM((STW,), jnp.int32),
        ],
    )
    return f(src, dst, ew)


def _prop_body(bpk, soff3, scnt3, gpad, s2out,
               gblock, moff, mcnt, ebig, espare, dchunk, stage, tmp16, acc,
               psem):
    c = lax.axis_index("c")
    s = lax.axis_index("s")
    nbase = s * NPT
    goff = c * GP
    lane = lax.iota(jnp.int32, 16)
    ssplat = jnp.full((16,), s, jnp.int32)

    def _lane_s(vec):
        tmp16[pl.ds(0, 16)] = vec
        return plsc.load_gather(tmp16, [ssplat])[0]

    pltpu.sync_copy(gpad.at[pl.ds(goff + nbase, NPT)],
                    acc.at[pl.ds(nbase, NPT)])

    @pl.when(s == 0)
    def _():
        pltpu.sync_copy(gpad.at[pl.ds(goff + NTILES * NPT, NTAIL)],
                        acc.at[pl.ds(NTILES * NPT, NTAIL)])
    plsc.subcore_barrier()

    for p in range(PASSES):
        b = p * 16 + s
        pltpu.sync_copy(gpad.at[pl.ds(goff + b * BB, BB)], gblock)
        pltpu.sync_copy(soff3.at[p], moff)
        pltpu.sync_copy(scnt3.at[p], mcnt)

        def _pre(t, _):
            offv = moff[t, pl.ds(0, 16)]
            r0 = _lane_s(offv)
            sa16 = r0 & -16
            ebase = t * STW + lax.shift_right_logical(sa16, 4) * 48
            pltpu.async_copy(bpk.at[pl.ds(ebase, 192)],
                             ebig.at[pl.ds(t * 192, 192)], psem)
            return _
        lax.fori_loop(0, 32, _pre, None)

        def _drain(t, _):
            pltpu.make_async_copy(bpk.at[pl.ds(0, 192)],
                                  ebig.at[pl.ds(0, 192)], psem).wait()
            return _
        lax.fori_loop(0, 32, _drain, None)

        def _seg(t, _):
            offv = moff[t, pl.ds(0, 16)]
            cntv = mcnt[t, pl.ds(0, 16)]
            r0 = _lane_s(offv)
            cnt = _lane_s(cntv)
            sa16 = r0 & -16
            head = r0 - sa16
            tot = head + cnt
            nch = lax.shift_right_logical(tot + 63, 6)
            ebase0 = t * STW + lax.shift_right_logical(sa16, 4) * 48

            @pl.when(cnt > 0)
            def _():
                for w in range(12):
                    espare[pl.ds(w * 16, 16)] = (
                        ebig[pl.ds(t * 192 + w * 16, 16)])

                def _chunk(i, _):
                    @pl.when(i > 0)
                    def _():
                        pltpu.sync_copy(
                            bpk.at[pl.ds(ebase0 + i * 192, 192)], espare)

                    def _grp(g, _):
                        srcv = espare[pl.ds(g * 48, 16)]
                        dstv = espare[pl.ds(g * 48 + 16, 16)]
                        ewv = plsc.bitcast(espare[pl.ds(g * 48 + 32, 16)],
                                           jnp.float32)
                        lr = i * 64 + g * 16 + lane
                        vm = (lr >= head) & (lr < tot)
                        ewv = jnp.where(vm, ewv, 0.0)
                        slv = jnp.clip(srcv - b * BB, 0, BB - 1)
                        dchunk[pl.ds(g * 16, 16)] = dstv
                        e0 = g * 16
                        for u in range(16):
                            wv = jnp.full((16,), ewv[u], jnp.float32)
                            iu = slv[u]
                            for q in range(H // 16):
                                sl = pl.ds(q * 16, 16)
                                stage[e0 + u, sl] = gblock[iu, sl] * wv
                        return _
                    lax.fori_loop(0, 4, _grp, None)
                    pltpu.sync_copy(stage, acc.at[dchunk], add=True)
                    return _
                lax.fori_loop(0, nch, _chunk, None)
            return _
        lax.fori_loop(0, 32, _seg, None)

    plsc.subcore_barrier()
    obase = c * N
    pltpu.sync_copy(acc.at[pl.ds(nbase, NPT)],
                    s2out.at[pl.ds(obase + nbase, NPT)])

    @pl.when(s == 0)
    def _():
        pltpu.sync_copy(acc.at[pl.ds(NTILES * NPT, NTAIL)],
                        s2out.at[pl.ds(obase + NTILES * NPT, NTAIL)])


def _propagate(bpk, soff3, scnt3, gpad):
    f = pl.kernel(
        _prop_body,
        out_type=jax.ShapeDtypeStruct((2 * N, H), jnp.float32),
        mesh=_sc_mesh(),
        compiler_params=pltpu.CompilerParams(needs_layout_passes=False),
        scratch_types=[
            pltpu.VMEM((BB, H), jnp.float32),
            pltpu.VMEM((32, 16), jnp.int32),
            pltpu.VMEM((32, 16), jnp.int32),
            pltpu.VMEM((32 * 192,), jnp.int32),
            pltpu.VMEM((192,), jnp.int32),
            pltpu.VMEM((64,), jnp.int32),
            pltpu.VMEM((64, H), jnp.float32),
            pltpu.VMEM((16,), jnp.int32),
            pltpu.VMEM_SHARED((N, H), jnp.float32),
            pltpu.SemaphoreType.DMA,
        ],
    )
    return f(bpk, soff3, scnt3, gpad)


def _mm1_body(x_ref, w_ref, part_ref, ga_ref, gb_ref, dinv_ref):
    deg = jnp.sum(part_ref[...], axis=1) + 1.0
    dinv = jnp.where(deg > 0, lax.rsqrt(jnp.maximum(deg, 1e-12)), 0.0)
    g = jnp.dot(x_ref[...], w_ref[...], precision=lax.Precision.HIGHEST,
                preferred_element_type=jnp.float32) * dinv[:, None]
    ga_ref[...] = g[:, :H]
    gb_ref[...] = g[:, H:]
    dinv_ref[...] = dinv[:, None]


def _mm1(x, W1, partials):
    return pl.pallas_call(
        _mm1_body,
        grid=(GRID,),
        in_specs=[
            pl.BlockSpec((MBLK, D), lambda i: (i, 0)),
            pl.BlockSpec((D, D), lambda i: (0, 0)),
            pl.BlockSpec((MBLK, 32), lambda i: (i, 0)),
        ],
        out_specs=[
            pl.BlockSpec((MBLK, H), lambda i: (i, 0)),
            pl.BlockSpec((MBLK, H), lambda i: (i, 0)),
            pl.BlockSpec((MBLK, 1), lambda i: (i, 0)),
        ],
        out_shape=[
            jax.ShapeDtypeStruct((N, H), jnp.float32),
            jax.ShapeDtypeStruct((N, H), jnp.float32),
            jax.ShapeDtypeStruct((N, 1), jnp.float32),
        ],
    )(x, W1, partials)


def _mm2_body(sa_ref, sb_ref, dinv_ref, b1a_ref, b1b_ref, w2a_ref, w2b_ref,
              ga_ref, gb_ref):
    dinv = dinv_ref[...]
    h1a = jax.nn.relu(sa_ref[...] * dinv + b1a_ref[...])
    h1b = jax.nn.relu(sb_ref[...] * dinv + b1b_ref[...])
    g = (jnp.dot(h1a, w2a_ref[...], precision=lax.Precision.HIGHEST,
                 preferred_element_type=jnp.float32)
         + jnp.dot(h1b, w2b_ref[...], precision=lax.Precision.HIGHEST,
                   preferred_element_type=jnp.float32)) * dinv
    ga_ref[...] = g[:, :H]
    gb_ref[...] = g[:, H:]


def _mm2(s2n, dinv, b1a, b1b, W2a, W2b):
    return pl.pallas_call(
        _mm2_body,
        grid=(GRID,),
        in_specs=[
            pl.BlockSpec((MBLK, H), lambda i: (i, 0)),
            pl.BlockSpec((MBLK, H), lambda i: (GRID + i, 0)),
            pl.BlockSpec((MBLK, 1), lambda i: (i, 0)),
            pl.BlockSpec((1, H), lambda i: (0, 0)),
            pl.BlockSpec((1, H), lambda i: (0, 0)),
            pl.BlockSpec((H, D), lambda i: (0, 0)),
            pl.BlockSpec((H, D), lambda i: (0, 0)),
        ],
        out_specs=[
            pl.BlockSpec((MBLK, H), lambda i: (i, 0)),
            pl.BlockSpec((MBLK, H), lambda i: (i, 0)),
        ],
        out_shape=[
            jax.ShapeDtypeStruct((N, H), jnp.float32),
            jax.ShapeDtypeStruct((N, H), jnp.float32),
        ],
    )(s2n, s2n, dinv, b1a, b1b, W2a, W2b)


def _mm3_body(sa_ref, sb_ref, dinv_ref, b2a_ref, b2b_ref, w3a_ref, w3b_ref,
              b3_ref, out_ref):
    dinv = dinv_ref[...]
    h2a = sa_ref[...] * dinv + b2a_ref[...]
    h2b = sb_ref[...] * dinv + b2b_ref[...]
    out_ref[...] = (jnp.dot(h2a, w3a_ref[...], precision=lax.Precision.HIGHEST,
                            preferred_element_type=jnp.float32)
                    + jnp.dot(h2b, w3b_ref[...],
                              precision=lax.Precision.HIGHEST,
                              preferred_element_type=jnp.float32)
                    + b3_ref[...])


def _mm3(s2n, dinv, b2a, b2b, W3a, W3b, b3):
    return pl.pallas_call(
        _mm3_body,
        grid=(GRID,),
        in_specs=[
            pl.BlockSpec((MBLK, H), lambda i: (i, 0)),
            pl.BlockSpec((MBLK, H), lambda i: (GRID + i, 0)),
            pl.BlockSpec((MBLK, 1), lambda i: (i, 0)),
            pl.BlockSpec((1, H), lambda i: (0, 0)),
            pl.BlockSpec((1, H), lambda i: (0, 0)),
            pl.BlockSpec((H, D), lambda i: (0, 0)),
            pl.BlockSpec((H, D), lambda i: (0, 0)),
            pl.BlockSpec((1, D), lambda i: (0, 0)),
        ],
        out_specs=pl.BlockSpec((MBLK, D), lambda i: (i, 0)),
        out_shape=jax.ShapeDtypeStruct((N, D), jnp.float32),
    )(s2n, s2n, dinv, b2a, b2b, W3a, W3b, b3)


def _gpad(ga, gb):
    return jnp.pad(jnp.stack([ga, gb]),
                   ((0, 0), (0, GP - N), (0, 0))).reshape(2 * GP, H)


@jax.jit
def kernel(x, edge_index, edge_attr, W1, b1, W2, b2, W3, b3):
    src = edge_index[0]
    dst = edge_index[1]

    degp, bpk, soff, scnt = _prep(src, dst, edge_attr)
    soff3 = soff.reshape(32, PASSES, 16).transpose(1, 0, 2)
    scnt3 = scnt.reshape(32, PASSES, 16).transpose(1, 0, 2)

    ga, gb, dinv = _mm1(x, W1, degp.reshape(32, N).T)
    s1 = _propagate(bpk, soff3, scnt3, _gpad(ga, gb))

    ga2, gb2 = _mm2(s1, dinv, b1[:H].reshape(1, H), b1[H:].reshape(1, H),
                    W2[:H], W2[H:])
    s2 = _propagate(bpk, soff3, scnt3, _gpad(ga2, gb2))

    return _mm3(s2, dinv, b2[:H].reshape(1, H), b2[H:].reshape(1, H),
                W3[:H], W3[H:], b3.reshape(1, D))


# speedup vs baseline: 2.3583x; 2.3583x over previous
import functools

import jax
import jax.numpy as jnp
from jax import lax
from jax.experimental import pallas as pl
from jax.experimental.pallas import tpu as pltpu
from jax.experimental.pallas import tpu_sc as plsc

N = 10000
E = 160000
D = 256
H = 128

NTILES = 16
ECHUNK = 128
EROWS = 1280
EPAD = EROWS * ECHUNK
ROWS_T = EROWS // NTILES
NPT = 624
NTAIL = N - NTILES * NPT

DEG_PER_TILE = E // 32
DEG_CHUNKS = (DEG_PER_TILE + 15) // 16
DEG_BUF = DEG_CHUNKS * 16

MBLK = 1000
GRID = N // MBLK


def _sc_mesh():
    return plsc.VectorSubcoreMesh(core_axis_name="c", subcore_axis_name="s")


def _deg_body(dst_hbm, ew_hbm, out_hbm, dstbuf, ewbuf, partial):
    c = lax.axis_index("c")
    s = lax.axis_index("s")
    wid = s * 2 + c
    base = wid * DEG_PER_TILE

    zi = jnp.zeros((16,), jnp.int32)
    zf = jnp.zeros((16,), jnp.float32)
    dstbuf[pl.ds(DEG_BUF - 16, 16)] = zi
    ewbuf[pl.ds(DEG_BUF - 16, 16)] = zf
    pltpu.sync_copy(dst_hbm.at[pl.ds(base, DEG_PER_TILE)],
                    dstbuf.at[pl.ds(0, DEG_PER_TILE)])
    pltpu.sync_copy(ew_hbm.at[pl.ds(base, DEG_PER_TILE)],
                    ewbuf.at[pl.ds(0, DEG_PER_TILE)])

    def _zero(i, _):
        partial[pl.ds(i * 16, 16)] = zf
        return _
    lax.fori_loop(0, N // 16, _zero, None)

    lane = lax.iota(jnp.int32, 16)

    def _scat(k, _):
        idx = dstbuf[pl.ds(k * 16, 16)]
        val = ewbuf[pl.ds(k * 16, 16)]
        mask = (k * 16 + lane) < DEG_PER_TILE
        plsc.addupdate_scatter(partial, [idx], val, mask=mask)
        return _
    lax.fori_loop(0, DEG_CHUNKS, _scat, None)

    pltpu.sync_copy(partial, out_hbm.at[pl.ds(wid * N, N)])


def _deg_partials(dst, ew):
    f = pl.kernel(
        _deg_body,
        out_type=jax.ShapeDtypeStruct((32 * N,), jnp.float32),
        mesh=_sc_mesh(),
        compiler_params=pltpu.CompilerParams(needs_layout_passes=False),
        scratch_types=[
            pltpu.VMEM((DEG_BUF,), jnp.int32),
            pltpu.VMEM((DEG_BUF,), jnp.float32),
            pltpu.VMEM((N,), jnp.float32),
        ],
    )
    return f(dst, ew)


def _prop_body(packed, ga, gb, sa, sb, ibuf0, ibuf1, rows0, rows1, acc,
               gsem0, gsem1, isem0, isem1):
    c = lax.axis_index("c")
    s = lax.axis_index("s")
    rbase = s * ROWS_T
    nbase = s * NPT

    def _scale(buf, ibuf):
        def _grp(t, _):
            ewv = plsc.bitcast(ibuf[2, pl.ds(t * 16, 16)], jnp.float32)
            for u in range(16):
                i = t * 16 + u
                wv = jnp.full((16,), ewv[u], jnp.float32)
                for q in range(H // 16):
                    sl = pl.ds(q * 16, 16)
                    buf[i, sl] = buf[i, sl] * wv
            return _
        lax.fori_loop(0, ECHUNK // 16, _grp, None)

    def _half(g_r, out_r):
        pltpu.sync_copy(g_r.at[pl.ds(nbase, NPT)], acc.at[pl.ds(nbase, NPT)])

        @pl.when(s == 0)
        def _():
            pltpu.sync_copy(g_r.at[pl.ds(NTILES * NPT, NTAIL)],
                            acc.at[pl.ds(NTILES * NPT, NTAIL)])
        plsc.subcore_barrier()

        pltpu.sync_copy(packed.at[rbase], ibuf0)
        pltpu.async_copy(packed.at[rbase + 1], ibuf1, isem1)
        pltpu.async_copy(g_r.at[ibuf0.at[0]], rows0, gsem0)
        last = ROWS_T // 2 - 1

        def _pair(k, _):
            j0 = rbase + 2 * k
            j1 = j0 + 1
            pltpu.make_async_copy(g_r.at[ibuf0.at[0]], rows0, gsem0).wait()
            pltpu.make_async_copy(packed.at[j1], ibuf1, isem1).wait()
            pltpu.async_copy(g_r.at[ibuf1.at[0]], rows1, gsem1)
            _scale(rows0, ibuf0)
            pltpu.sync_copy(rows0, acc.at[ibuf0.at[1]], add=True)

            @pl.when(k < last)
            def _():
                pltpu.async_copy(packed.at[j0 + 2], ibuf0, isem0)

            pltpu.make_async_copy(g_r.at[ibuf1.at[0]], rows1, gsem1).wait()

            @pl.when(k < last)
            def _():
                pltpu.make_async_copy(packed.at[j0 + 2], ibuf0, isem0).wait()
                pltpu.async_copy(g_r.at[ibuf0.at[0]], rows0, gsem0)
            _scale(rows1, ibuf1)
            pltpu.sync_copy(rows1, acc.at[ibuf1.at[1]], add=True)

            @pl.when(k < last)
            def _():
                pltpu.async_copy(packed.at[j1 + 2], ibuf1, isem1)
            return _
        lax.fori_loop(0, ROWS_T // 2, _pair, None)

        plsc.subcore_barrier()
        pltpu.sync_copy(acc.at[pl.ds(nbase, NPT)], out_r.at[pl.ds(nbase, NPT)])

        @pl.when(s == 0)
        def _():
            pltpu.sync_copy(acc.at[pl.ds(NTILES * NPT, NTAIL)],
                            out_r.at[pl.ds(NTILES * NPT, NTAIL)])

    @pl.when(c == 0)
    def _():
        _half(ga, sa)

    @pl.when(c == 1)
    def _():
        _half(gb, sb)


def _propagate(packed, ga, gb):
    f = pl.kernel(
        _prop_body,
        out_type=[jax.ShapeDtypeStruct((N, H), jnp.float32),
                  jax.ShapeDtypeStruct((N, H), jnp.float32)],
        mesh=_sc_mesh(),
        compiler_params=pltpu.CompilerParams(needs_layout_passes=False),
        scratch_types=[
            pltpu.VMEM((3, ECHUNK), jnp.int32),
            pltpu.VMEM((3, ECHUNK), jnp.int32),
            pltpu.VMEM((ECHUNK, H), jnp.float32),
            pltpu.VMEM((ECHUNK, H), jnp.float32),
            pltpu.VMEM_SHARED((N, H), jnp.float32),
            pltpu.SemaphoreType.DMA,
            pltpu.SemaphoreType.DMA,
            pltpu.SemaphoreType.DMA,
            pltpu.SemaphoreType.DMA,
        ],
    )
    return f(packed, ga, gb)


def _mm1_body(x_ref, w_ref, part_ref, ga_ref, gb_ref, dinv_ref):
    deg = jnp.sum(part_ref[...], axis=1) + 1.0
    dinv = jnp.where(deg > 0, lax.rsqrt(jnp.maximum(deg, 1e-12)), 0.0)
    g = jnp.dot(x_ref[...], w_ref[...], precision=lax.Precision.HIGHEST,
                preferred_element_type=jnp.float32) * dinv[:, None]
    ga_ref[...] = g[:, :H]
    gb_ref[...] = g[:, H:]
    dinv_ref[...] = dinv[:, None]


def _mm1(x, W1, partials):
    return pl.pallas_call(
        _mm1_body,
        grid=(GRID,),
        in_specs=[
            pl.BlockSpec((MBLK, D), lambda i: (i, 0)),
            pl.BlockSpec((D, D), lambda i: (0, 0)),
            pl.BlockSpec((MBLK, 32), lambda i: (i, 0)),
        ],
        out_specs=[
            pl.BlockSpec((MBLK, H), lambda i: (i, 0)),
            pl.BlockSpec((MBLK, H), lambda i: (i, 0)),
            pl.BlockSpec((MBLK, 1), lambda i: (i, 0)),
        ],
        out_shape=[
            jax.ShapeDtypeStruct((N, H), jnp.float32),
            jax.ShapeDtypeStruct((N, H), jnp.float32),
            jax.ShapeDtypeStruct((N, 1), jnp.float32),
        ],
    )(x, W1, partials)


def _mm2_body(sa_ref, sb_ref, dinv_ref, b1a_ref, b1b_ref, w2a_ref, w2b_ref,
              ga_ref, gb_ref):
    dinv = dinv_ref[...]
    h1a = jax.nn.relu(sa_ref[...] * dinv + b1a_ref[...])
    h1b = jax.nn.relu(sb_ref[...] * dinv + b1b_ref[...])
    g = (jnp.dot(h1a, w2a_ref[...], precision=lax.Precision.HIGHEST,
                 preferred_element_type=jnp.float32)
         + jnp.dot(h1b, w2b_ref[...], precision=lax.Precision.HIGHEST,
                   preferred_element_type=jnp.float32)) * dinv
    ga_ref[...] = g[:, :H]
    gb_ref[...] = g[:, H:]


def _mm2(sa, sb, dinv, b1a, b1b, W2a, W2b):
    return pl.pallas_call(
        _mm2_body,
        grid=(GRID,),
        in_specs=[
            pl.BlockSpec((MBLK, H), lambda i: (i, 0)),
            pl.BlockSpec((MBLK, H), lambda i: (i, 0)),
            pl.BlockSpec((MBLK, 1), lambda i: (i, 0)),
            pl.BlockSpec((1, H), lambda i: (0, 0)),
            pl.BlockSpec((1, H), lambda i: (0, 0)),
            pl.BlockSpec((H, D), lambda i: (0, 0)),
            pl.BlockSpec((H, D), lambda i: (0, 0)),
        ],
        out_specs=[
            pl.BlockSpec((MBLK, H), lambda i: (i, 0)),
            pl.BlockSpec((MBLK, H), lambda i: (i, 0)),
        ],
        out_shape=[
            jax.ShapeDtypeStruct((N, H), jnp.float32),
            jax.ShapeDtypeStruct((N, H), jnp.float32),
        ],
    )(sa, sb, dinv, b1a, b1b, W2a, W2b)


def _mm3_body(sa_ref, sb_ref, dinv_ref, b2a_ref, b2b_ref, w3a_ref, w3b_ref,
              b3_ref, out_ref):
    dinv = dinv_ref[...]
    h2a = sa_ref[...] * dinv + b2a_ref[...]
    h2b = sb_ref[...] * dinv + b2b_ref[...]
    out_ref[...] = (jnp.dot(h2a, w3a_ref[...], precision=lax.Precision.HIGHEST,
                            preferred_element_type=jnp.float32)
                    + jnp.dot(h2b, w3b_ref[...],
                              precision=lax.Precision.HIGHEST,
                              preferred_element_type=jnp.float32)
                    + b3_ref[...])


def _mm3(sa, sb, dinv, b2a, b2b, W3a, W3b, b3):
    return pl.pallas_call(
        _mm3_body,
        grid=(GRID,),
        in_specs=[
            pl.BlockSpec((MBLK, H), lambda i: (i, 0)),
            pl.BlockSpec((MBLK, H), lambda i: (i, 0)),
            pl.BlockSpec((MBLK, 1), lambda i: (i, 0)),
            pl.BlockSpec((1, H), lambda i: (0, 0)),
            pl.BlockSpec((1, H), lambda i: (0, 0)),
            pl.BlockSpec((H, D), lambda i: (0, 0)),
            pl.BlockSpec((H, D), lambda i: (0, 0)),
            pl.BlockSpec((1, D), lambda i: (0, 0)),
        ],
        out_specs=pl.BlockSpec((MBLK, D), lambda i: (i, 0)),
        out_shape=jax.ShapeDtypeStruct((N, D), jnp.float32),
    )(sa, sb, dinv, b2a, b2b, W3a, W3b, b3)


@jax.jit
def kernel(x, edge_index, edge_attr, W1, b1, W2, b2, W3, b3):
    src = edge_index[0]
    dst = edge_index[1]
    pad = EPAD - E
    src2 = jnp.pad(src, (0, pad)).reshape(EROWS, ECHUNK)
    dst2 = jnp.pad(dst, (0, pad)).reshape(EROWS, ECHUNK)
    ew2 = lax.bitcast_convert_type(
        jnp.pad(edge_attr, (0, pad)).reshape(EROWS, ECHUNK), jnp.int32)
    packed = jnp.stack([src2, dst2, ew2], axis=1)

    partials = _deg_partials(dst, edge_attr)

    ga, gb, dinv = _mm1(x, W1, partials.reshape(32, N).T)
    sa, sb = _propagate(packed, ga, gb)

    ga2, gb2 = _mm2(sa, sb, dinv, b1[:H].reshape(1, H), b1[H:].reshape(1, H),
                    W2[:H], W2[H:])
    sa2, sb2 = _propagate(packed, ga2, gb2)

    return _mm3(sa2, sb2, dinv, b2[:H].reshape(1, H), b2[H:].reshape(1, H),
                W3[:H], W3[H:], b3.reshape(1, D))
